# baseline (speedup 1.0000x reference)
import functools

import jax
import jax.numpy as jnp
from jax import lax
from jax.experimental import pallas as pl
from jax.experimental.pallas import tpu as pltpu
from jax.experimental.pallas import tpu_sc as plsc

NC = 2
NS = 16
NW = NC * NS

CHUNK = 128
NBUF = 8
DEPTH = 4


def _make_gather(total, d):
    assert total % (NW * CHUNK) == 0
    per_w = total // NW
    n_chunks = per_w // CHUNK
    assert n_chunks % NBUF == 0
    mesh = plsc.VectorSubcoreMesh(core_axis_name="c", subcore_axis_name="s")

    @functools.partial(
        pl.kernel,
        mesh=mesh,
        out_type=jax.ShapeDtypeStruct((total, d), jnp.float32),
        compiler_params=pltpu.CompilerParams(use_tc_tiling_on_sc=False),
        scratch_types=[
            pltpu.VMEM((n_chunks, CHUNK), jnp.int32),
            pltpu.VMEM((NBUF, CHUNK, d), jnp.float32),
            pltpu.SemaphoreType.DMA((NBUF,)),
            pltpu.SemaphoreType.DMA((NBUF,)),
        ],
    )
    def gather_kernel(table_hbm, idx_hbm, out_hbm, idx_v, rows_v, gsem, osem):
        wid = lax.axis_index("s") * NC + lax.axis_index("c")
        row0 = wid * per_w

        pltpu.sync_copy(idx_hbm.at[wid], idx_v)

        def start_gather(j, b):
            pltpu.make_async_copy(
                table_hbm.at[idx_v.at[j]], rows_v.at[b], gsem.at[b]
            ).start()

        def wait_gather(j, b):
            pltpu.make_async_copy(
                table_hbm.at[idx_v.at[j]], rows_v.at[b], gsem.at[b]
            ).wait()

        def start_out(j, b):
            pltpu.make_async_copy(
                rows_v.at[b], out_hbm.at[pl.ds(row0 + j * CHUNK, CHUNK)],
                osem.at[b],
            ).start()

        def wait_out(j, b):
            pltpu.make_async_copy(
                rows_v.at[b], out_hbm.at[pl.ds(row0 + j * CHUNK, CHUNK)],
                osem.at[b],
            ).wait()

        for g in range(DEPTH):
            start_gather(g, g)

        def outer(i, carry):
            j0 = i * NBUF
            for b in range(NBUF):
                j = j0 + b
                wait_gather(j, b)
                start_out(j, b)
                g = j + DEPTH
                bg = (b + DEPTH) % NBUF

                @pl.when(jnp.logical_and(g < n_chunks, g >= NBUF))
                def _():
                    wait_out(g - NBUF, bg)

                @pl.when(g < n_chunks)
                def _():
                    start_gather(g, bg)
            return carry

        lax.fori_loop(0, n_chunks // NBUF, outer, 0)

        for b in range(NBUF):
            wait_out(n_chunks - NBUF + b, b)

    return gather_kernel


@jax.jit
def kernel(x, weight):
    batch, n_fields = x.shape
    total = batch * n_fields
    d = weight.shape[1]
    per_w = total // NW
    idx = x.reshape(NW, per_w // CHUNK, CHUNK).astype(jnp.int32)
    out = _make_gather(total, d)(weight, idx)
    return out.reshape(batch, n_fields, d)
